# baseline (speedup 1.0000x reference)
import functools

import jax
import jax.numpy as jnp
from jax import lax
from jax.experimental import pallas as pl
from jax.experimental.pallas import tpu as pltpu
from jax.experimental.pallas import tpu_sc as plsc

BATCH = 16384
D = 16
ROWS_PER_LINE = 8
LINE = ROWS_PER_LINE * D
NC = 2
NS = 16
NW = NC * NS
BW = BATCH // NW
CHUNK = 128
NCHUNK = BW // CHUNK

_mesh = plsc.VectorSubcoreMesh(core_axis_name="c", subcore_axis_name="s")


@functools.partial(
    pl.kernel,
    mesh=_mesh,
    out_type=jax.ShapeDtypeStruct((BATCH,), jnp.float32),
    scratch_types=[
        pltpu.VMEM((NCHUNK, CHUNK), jnp.int32),
        pltpu.VMEM((NCHUNK, CHUNK), jnp.int32),
        pltpu.VMEM((NCHUNK, CHUNK), jnp.int32),
        pltpu.VMEM((NCHUNK, CHUNK), jnp.int32),
        pltpu.VMEM((CHUNK, LINE), jnp.float32),
        pltpu.VMEM((CHUNK, LINE), jnp.float32),
        pltpu.VMEM((BW,), jnp.float32),
        pltpu.SemaphoreType.DMA,
    ],
    compiler_params=pltpu.CompilerParams(needs_layout_passes=False),
)
def _mf_sc(xu_hbm, xv_hbm, uemb_hbm, vemb_hbm, out_hbm,
           xu_v, xv_v, xlu_v, xlv_v, ulines, vlines, out_v, sem):
    wid = lax.axis_index("s") * NC + lax.axis_index("c")
    base = wid * BW

    pltpu.sync_copy(xu_hbm.at[pl.ds(wid * NCHUNK, NCHUNK), :], xu_v)
    pltpu.sync_copy(xv_hbm.at[pl.ds(wid * NCHUNK, NCHUNK), :], xv_v)

    for j in range(NCHUNK):
        for t in range(CHUNK // 16):
            s = pl.ds(t * 16, 16)
            xlu_v[j, s] = lax.shift_right_logical(xu_v[j, s], 3)
            xlv_v[j, s] = lax.shift_right_logical(xv_v[j, s], 3)

    lane_ids = lax.iota(jnp.int32, 16)
    for j in range(NCHUNK):
        cu = pltpu.async_copy(uemb_hbm.at[xlu_v.at[j]], ulines, sem)
        cv = pltpu.async_copy(vemb_hbm.at[xlv_v.at[j]], vlines, sem)
        cu.wait()
        cv.wait()
        for b in range(CHUNK // 16):
            s = pl.ds(b * 16, 16)
            rows16 = b * 16 + lane_ids
            offu = (xu_v[j, s] & 7) * D
            offv = (xv_v[j, s] & 7) * D
            acc = jnp.zeros((16,), jnp.float32)
            for d in range(D):
                acc = acc + (plsc.load_gather(ulines, [rows16, offu + d])
                             * plsc.load_gather(vlines, [rows16, offv + d]))
            out_v[pl.ds(j * CHUNK + b * 16, 16)] = acc

    pltpu.sync_copy(out_v, out_hbm.at[pl.ds(base, BW)])


def kernel(X, user_emb, item_emb):
    xu = X[:, 0].reshape(NW * NCHUNK, CHUNK)
    xv = X[:, 1].reshape(NW * NCHUNK, CHUNK)
    ue = user_emb.reshape(-1, LINE)
    ve = item_emb.reshape(-1, LINE)
    out = _mf_sc(xu, xv, ue, ve)
    return out.reshape(BATCH, 1)
